# baseline (speedup 1.0000x reference)
import functools

import jax
import jax.numpy as jnp
from jax import lax
from jax.experimental import pallas as pl
from jax.experimental.pallas import tpu as pltpu
from jax.experimental.pallas import tpu_sc as plsc

N_FIELDS = 26
N_DENSE = 13
VOCAB = 100000
DIM = 32
BATCH = 16384
OUT_COLS = N_FIELDS * DIM + N_DENSE

NC, NS = 2, 16
NW = NC * NS
ROWS_PER_W = BATCH // NW
R = 128
N_CHUNKS = ROWS_PER_W // R

_mesh = plsc.VectorSubcoreMesh(
    core_axis_name="c", subcore_axis_name="s", num_cores=NC, num_subcores=NS
)


@functools.partial(
    pl.kernel,
    out_type=jax.ShapeDtypeStruct((BATCH, OUT_COLS), jnp.float32),
    mesh=_mesh,
    scratch_types=[
        pltpu.VMEM((N_FIELDS, R), jnp.int32),
        pltpu.VMEM((N_FIELDS, R, DIM), jnp.float32),
        pltpu.VMEM((R, N_DENSE), jnp.float32),
        pltpu.SemaphoreType.DMA,
    ],
    compiler_params=pltpu.CompilerParams(use_tc_tiling_on_sc=False),
)
def _combine(idx_hbm, dense_hbm, tbl_hbm, out_hbm, idx_v, emb_v, dense_v, sem):
    wid = lax.axis_index("s") * NC + lax.axis_index("c")

    @pl.loop(0, N_CHUNKS)
    def _chunk(c):
        base = wid * ROWS_PER_W + c * R
        pltpu.sync_copy(idx_hbm.at[:, pl.ds(base, R)], idx_v)
        descs = [
            pltpu.async_copy(tbl_hbm.at[idx_v.at[f]], emb_v.at[f], sem)
            for f in range(N_FIELDS)
        ]
        for d in descs:
            d.wait()
        for f in range(N_FIELDS):
            pltpu.sync_copy(
                emb_v.at[f], out_hbm.at[pl.ds(base, R), pl.ds(f * DIM, DIM)]
            )
        pltpu.sync_copy(dense_hbm.at[pl.ds(base, R)], dense_v)
        pltpu.sync_copy(
            dense_v, out_hbm.at[pl.ds(base, R), pl.ds(N_FIELDS * DIM, N_DENSE)]
        )


def kernel(indices, dense, tables):
    flat_idx = indices + (jnp.arange(N_FIELDS, dtype=jnp.int32) * VOCAB)[:, None]
    dense_t = dense.T
    flat_tbl = tables.reshape(N_FIELDS * VOCAB, DIM)
    return _combine(flat_idx, dense_t, flat_tbl)
